# Initial kernel scaffold; baseline (speedup 1.0000x reference)
import jax
import jax.numpy as jnp
from jax import lax
from jax.experimental import pallas as pl
from jax.experimental.pallas import tpu as pltpu
from jax.experimental.pallas import tpu_sc as plsc

N = 10000
E = 320000
D = 128

NC = 2
NS = 16
NW = NC * NS
EPW = E // NW
C = 80
NCHUNK = EPW // C
NP = 10240
RPS = NP // NS
DW = 16

_MESH = plsc.VectorSubcoreMesh(core_axis_name="c", subcore_axis_name="s")


def _deg_body(dst_hbm, tab_hbm, ones_v, zbuf_v, idx0, idx1, tab_sh, lsem0, lsem1):
    cid = lax.axis_index("c")
    sid = lax.axis_index("s")
    base = (cid * NS + sid) * EPW

    @pl.loop(0, C)
    def _(r):
        ones_v[r, :] = jnp.full((DW,), 1.0, jnp.float32)

    @pl.loop(0, 128)
    def _(r):
        zbuf_v[r, :] = jnp.zeros((DW,), jnp.float32)

    @pl.loop(0, RPS // 128)
    def _(t):
        pltpu.sync_copy(zbuf_v, tab_sh.at[pl.ds(sid * RPS + t * 128, 128)])

    plsc.subcore_barrier()

    pltpu.sync_copy(dst_hbm.at[pl.ds(base, C)], idx0)

    @pl.loop(0, NCHUNK - 1, step=2)
    def _(j):
        l1 = pltpu.async_copy(dst_hbm.at[pl.ds(base + (j + 1) * C, C)], idx1, lsem1)
        pltpu.sync_copy(ones_v, tab_sh.at[idx0], add=True)
        l1.wait()
        pltpu.async_copy(dst_hbm.at[pl.ds(base + (j + 2) * C, C)], idx0, lsem0)
        pltpu.sync_copy(ones_v, tab_sh.at[idx1], add=True)
        pltpu.make_async_copy(dst_hbm.at[pl.ds(base, C)], idx0, lsem0).wait()

    pltpu.sync_copy(ones_v, tab_sh.at[idx0], add=True)

    plsc.subcore_barrier()
    pltpu.sync_copy(tab_sh.at[pl.ds(sid * RPS, RPS)],
                    tab_hbm.at[pl.ds(cid * NP + sid * RPS, RPS)])


@jax.jit
def _deg_call(dst):
    k = pl.kernel(
        _deg_body,
        out_type=jax.ShapeDtypeStruct((NC * NP, DW), jnp.float32),
        mesh=_MESH,
        scratch_types=[
            pltpu.VMEM((C, DW), jnp.float32),
            pltpu.VMEM((128, DW), jnp.float32),
            pltpu.VMEM((C,), jnp.int32),
            pltpu.VMEM((C,), jnp.int32),
            pltpu.SemaphoreType.DMA,
            pltpu.SemaphoreType.DMA,
        ],
    )
    return k(dst)


def _agg_body(src_hbm, dst_hbm, y_hbm, agg_hbm,
              is0, is1, id0, id1, rows0, rows1, agg_sh, gsem0, gsem1):
    cid = lax.axis_index("c")
    sid = lax.axis_index("s")
    base = (cid * NS + sid) * EPW

    @pl.loop(0, C)
    def _(r):
        @pl.loop(0, D // 16)
        def _(q):
            rows0[r, pl.ds(q * 16, 16)] = jnp.zeros((16,), jnp.float32)

    @pl.loop(0, RPS // C)
    def _(t):
        pltpu.sync_copy(rows0, agg_sh.at[pl.ds(sid * RPS + t * C, C)])

    plsc.subcore_barrier()

    pltpu.sync_copy(src_hbm.at[pl.ds(base, C)], is0)
    pltpu.sync_copy(dst_hbm.at[pl.ds(base, C)], id0)
    pltpu.async_copy(y_hbm.at[is0], rows0, gsem0)

    @pl.loop(0, NCHUNK - 1, step=2)
    def _(j):
        pltpu.sync_copy(src_hbm.at[pl.ds(base + (j + 1) * C, C)], is1)
        pltpu.sync_copy(dst_hbm.at[pl.ds(base + (j + 1) * C, C)], id1)
        g1 = pltpu.async_copy(y_hbm.at[is1], rows1, gsem1)
        pltpu.make_async_copy(y_hbm.at[is0], rows0, gsem0).wait()
        pltpu.sync_copy(rows0, agg_sh.at[id0], add=True)
        pltpu.sync_copy(src_hbm.at[pl.ds(base + (j + 2) * C, C)], is0)
        pltpu.sync_copy(dst_hbm.at[pl.ds(base + (j + 2) * C, C)], id0)
        pltpu.async_copy(y_hbm.at[is0], rows0, gsem0)
        g1.wait()
        pltpu.sync_copy(rows1, agg_sh.at[id1], add=True)

    pltpu.make_async_copy(y_hbm.at[is0], rows0, gsem0).wait()
    pltpu.sync_copy(rows0, agg_sh.at[id0], add=True)

    plsc.subcore_barrier()
    pltpu.sync_copy(agg_sh.at[pl.ds(sid * RPS, RPS)],
                    agg_hbm.at[pl.ds(cid * NP + sid * RPS, RPS)])


@jax.jit
def _agg_call(src, dst, y):
    k = pl.kernel(
        _agg_body,
        out_type=jax.ShapeDtypeStruct((NC * NP, D), jnp.float32),
        mesh=_MESH,
        scratch_types=[
            pltpu.VMEM((C,), jnp.int32),
            pltpu.VMEM((C,), jnp.int32),
            pltpu.VMEM((C,), jnp.int32),
            pltpu.VMEM((C,), jnp.int32),
            pltpu.VMEM((C, D), jnp.float32),
            pltpu.VMEM((C, D), jnp.float32),
            pltpu.VMEM_SHARED((NP, D), jnp.float32),
            pltpu.SemaphoreType.DMA,
            pltpu.SemaphoreType.DMA,
        ],
    )
    return k(src, dst, y)


_RB = 1000


def _mm_body(x_ref, w_ref, o_ref):
    o_ref[...] = jnp.dot(x_ref[...], w_ref[...],
                         preferred_element_type=jnp.float32)


@jax.jit
def _mm_call(x, W):
    return pl.pallas_call(
        _mm_body,
        grid=(N // _RB,),
        in_specs=[
            pl.BlockSpec((_RB, D), lambda i: (i, 0)),
            pl.BlockSpec((D, D), lambda i: (0, 0)),
        ],
        out_specs=pl.BlockSpec((_RB, D), lambda i: (i, 0)),
        out_shape=jax.ShapeDtypeStruct((N, D), jnp.float32),
    )(x, W)


def _scale_body(xw_ref, t_ref, o_ref):
    deg = t_ref[0, :, 0] + t_ref[1, :, 0] + 1.0
    dis = lax.rsqrt(deg)
    o_ref[...] = xw_ref[...] * dis[:, None]


@jax.jit
def _scale_call(xw, tab3):
    return pl.pallas_call(
        _scale_body,
        grid=(N // _RB,),
        in_specs=[
            pl.BlockSpec((_RB, D), lambda i: (i, 0)),
            pl.BlockSpec((NC, _RB, DW), lambda i: (0, i, 0)),
        ],
        out_specs=pl.BlockSpec((_RB, D), lambda i: (i, 0)),
        out_shape=jax.ShapeDtypeStruct((N, D), jnp.float32),
    )(xw, tab3)


def _final_body(a_ref, y_ref, t_ref, b_ref, o_ref):
    deg = t_ref[0, :, 0] + t_ref[1, :, 0] + 1.0
    dis = lax.rsqrt(deg)
    acc = a_ref[0] + a_ref[1] + y_ref[...]
    o_ref[...] = acc * dis[:, None] + b_ref[...][None, :]


@jax.jit
def _final_call(agg3, y, tab3, b):
    return pl.pallas_call(
        _final_body,
        grid=(N // _RB,),
        in_specs=[
            pl.BlockSpec((NC, _RB, D), lambda i: (0, i, 0)),
            pl.BlockSpec((_RB, D), lambda i: (i, 0)),
            pl.BlockSpec((NC, _RB, DW), lambda i: (0, i, 0)),
            pl.BlockSpec((D,), lambda i: (0,)),
        ],
        out_specs=pl.BlockSpec((_RB, D), lambda i: (i, 0)),
        out_shape=jax.ShapeDtypeStruct((N, D), jnp.float32),
    )(agg3, y, tab3, b)


def kernel(x, edge_index, W, b):
    src = edge_index[0]
    dst = edge_index[1]
    tab = _deg_call(dst)
    xw = _mm_call(x, W)
    tab3 = tab.reshape(NC, NP, DW)
    y = _scale_call(xw, tab3)
    agg = _agg_call(src, dst, y)
    out = _final_call(agg.reshape(NC, NP, D), y, tab3, b)
    return out


# speedup vs baseline: 26.2085x; 26.2085x over previous
import dataclasses

import jax
import jax.numpy as jnp
from jax import lax
from jax.experimental import pallas as pl
from jax.experimental.pallas import tpu as pltpu
from jax.experimental.pallas import tpu_sc as plsc

N = 10000
E = 320000
D = 128

NC = 2
NS = 16
NW = NC * NS
EPW = E // NW
C = 80
NCHUNK = EPW // C
NP = 10240
RPS = NP // NS
DW = 16

def _sc_params():
    cp = pltpu.CompilerParams()
    if "needs_layout_passes" in pltpu.CompilerParams.__dataclass_fields__:
        cp = dataclasses.replace(cp, needs_layout_passes=False)
    return cp


def _mesh():
    return plsc.VectorSubcoreMesh(core_axis_name="c", subcore_axis_name="s",
                                  num_cores=NC, num_subcores=NS)


def _deg_body(dst_hbm, deg_hbm, hist, idx0, idx1, lsem0, lsem1):
    cid = lax.axis_index("c")
    sid = lax.axis_index("s")
    w = cid * NS + sid
    base = w * EPW
    ones = jnp.full((16,), 1.0, jnp.float32)

    @pl.loop(0, N // 16)
    def _(i):
        hist[pl.ds(i * 16, 16)] = jnp.zeros((16,), jnp.float32)

    def scat(idx_buf):
        @pl.loop(0, C // 16)
        def _(g):
            plsc.addupdate_scatter(hist, [idx_buf[pl.ds(g * 16, 16)]], ones)

    pltpu.sync_copy(dst_hbm.at[pl.ds(base, C)], idx0)

    @pl.loop(0, NCHUNK - 1, step=2)
    def _(j):
        l1 = pltpu.async_copy(dst_hbm.at[pl.ds(base + (j + 1) * C, C)], idx1, lsem1)
        scat(idx0)
        l1.wait()
        pltpu.async_copy(dst_hbm.at[pl.ds(base + (j + 2) * C, C)], idx0, lsem0)
        scat(idx1)
        pltpu.make_async_copy(dst_hbm.at[pl.ds(base, C)], idx0, lsem0).wait()

    scat(idx0)
    pltpu.sync_copy(hist, deg_hbm.at[w])


@jax.jit
def _deg_call(dst):
    k = pl.kernel(
        _deg_body,
        out_type=jax.ShapeDtypeStruct((NW, N), jnp.float32),
        mesh=_mesh(),
        compiler_params=_sc_params(),
        scratch_types=[
            pltpu.VMEM((N,), jnp.float32),
            pltpu.VMEM((C,), jnp.int32),
            pltpu.VMEM((C,), jnp.int32),
            pltpu.SemaphoreType.DMA,
            pltpu.SemaphoreType.DMA,
        ],
    )
    return k(dst)


def _agg_body(src_hbm, dst_hbm, y_hbm, agg_hbm,
              is0, is1, id0, id1, rows0, rows1, agg_sh, gsem0, gsem1):
    cid = lax.axis_index("c")
    sid = lax.axis_index("s")
    base = (cid * NS + sid) * EPW

    @pl.loop(0, C)
    def _(r):
        @pl.loop(0, D // 16)
        def _(q):
            rows0[r, pl.ds(q * 16, 16)] = jnp.zeros((16,), jnp.float32)

    @pl.loop(0, RPS // C)
    def _(t):
        pltpu.sync_copy(rows0, agg_sh.at[pl.ds(sid * RPS + t * C, C)])

    plsc.subcore_barrier()

    pltpu.sync_copy(src_hbm.at[pl.ds(base, C)], is0)
    pltpu.sync_copy(dst_hbm.at[pl.ds(base, C)], id0)
    pltpu.async_copy(y_hbm.at[is0], rows0, gsem0)

    @pl.loop(0, NCHUNK - 1, step=2)
    def _(j):
        pltpu.sync_copy(src_hbm.at[pl.ds(base + (j + 1) * C, C)], is1)
        pltpu.sync_copy(dst_hbm.at[pl.ds(base + (j + 1) * C, C)], id1)
        g1 = pltpu.async_copy(y_hbm.at[is1], rows1, gsem1)
        pltpu.make_async_copy(y_hbm.at[is0], rows0, gsem0).wait()
        pltpu.sync_copy(rows0, agg_sh.at[id0], add=True)
        pltpu.sync_copy(src_hbm.at[pl.ds(base + (j + 2) * C, C)], is0)
        pltpu.sync_copy(dst_hbm.at[pl.ds(base + (j + 2) * C, C)], id0)
        pltpu.async_copy(y_hbm.at[is0], rows0, gsem0)
        g1.wait()
        pltpu.sync_copy(rows1, agg_sh.at[id1], add=True)

    pltpu.make_async_copy(y_hbm.at[is0], rows0, gsem0).wait()
    pltpu.sync_copy(rows0, agg_sh.at[id0], add=True)

    plsc.subcore_barrier()
    pltpu.sync_copy(agg_sh.at[pl.ds(sid * RPS, RPS)],
                    agg_hbm.at[pl.ds(cid * NP + sid * RPS, RPS)])


@jax.jit
def _agg_call(src, dst, y):
    k = pl.kernel(
        _agg_body,
        out_type=jax.ShapeDtypeStruct((NC * NP, D), jnp.float32),
        mesh=_mesh(),
        scratch_types=[
            pltpu.VMEM((C,), jnp.int32),
            pltpu.VMEM((C,), jnp.int32),
            pltpu.VMEM((C,), jnp.int32),
            pltpu.VMEM((C,), jnp.int32),
            pltpu.VMEM((C, D), jnp.float32),
            pltpu.VMEM((C, D), jnp.float32),
            pltpu.VMEM_SHARED((NP, D), jnp.float32),
            pltpu.SemaphoreType.DMA,
            pltpu.SemaphoreType.DMA,
        ],
    )
    return k(src, dst, y)


_RB = 1000


def _mm_body(x_ref, w_ref, o_ref):
    o_ref[...] = jnp.dot(x_ref[...], w_ref[...],
                         preferred_element_type=jnp.float32)


@jax.jit
def _mm_call(x, W):
    return pl.pallas_call(
        _mm_body,
        grid=(N // _RB,),
        in_specs=[
            pl.BlockSpec((_RB, D), lambda i: (i, 0)),
            pl.BlockSpec((D, D), lambda i: (0, 0)),
        ],
        out_specs=pl.BlockSpec((_RB, D), lambda i: (i, 0)),
        out_shape=jax.ShapeDtypeStruct((N, D), jnp.float32),
    )(x, W)


def _dis_body(t_ref, o_ref):
    deg = jnp.sum(t_ref[...], axis=0) + 1.0
    o_ref[...] = lax.rsqrt(deg)[:, None]


@jax.jit
def _dis_call(deg_rows):
    return pl.pallas_call(
        _dis_body,
        grid=(1,),
        in_specs=[pl.BlockSpec((NW, N), lambda i: (0, 0))],
        out_specs=pl.BlockSpec((N, 1), lambda i: (0, 0)),
        out_shape=jax.ShapeDtypeStruct((N, 1), jnp.float32),
    )(deg_rows)


def _scale_body(xw_ref, dis_ref, o_ref):
    o_ref[...] = xw_ref[...] * dis_ref[...]


@jax.jit
def _scale_call(xw, dis):
    return pl.pallas_call(
        _scale_body,
        grid=(N // _RB,),
        in_specs=[
            pl.BlockSpec((_RB, D), lambda i: (i, 0)),
            pl.BlockSpec((_RB, 1), lambda i: (i, 0)),
        ],
        out_specs=pl.BlockSpec((_RB, D), lambda i: (i, 0)),
        out_shape=jax.ShapeDtypeStruct((N, D), jnp.float32),
    )(xw, dis)


def _final_body(a_ref, y_ref, dis_ref, b_ref, o_ref):
    acc = a_ref[0] + a_ref[1] + y_ref[...]
    o_ref[...] = acc * dis_ref[...] + b_ref[...][None, :]


@jax.jit
def _final_call(agg3, y, dis, b):
    return pl.pallas_call(
        _final_body,
        grid=(N // _RB,),
        in_specs=[
            pl.BlockSpec((NC, _RB, D), lambda i: (0, i, 0)),
            pl.BlockSpec((_RB, D), lambda i: (i, 0)),
            pl.BlockSpec((_RB, 1), lambda i: (i, 0)),
            pl.BlockSpec((D,), lambda i: (0,)),
        ],
        out_specs=pl.BlockSpec((_RB, D), lambda i: (i, 0)),
        out_shape=jax.ShapeDtypeStruct((N, D), jnp.float32),
    )(agg3, y, dis, b)


def kernel(x, edge_index, W, b):
    src = edge_index[0]
    dst = edge_index[1]
    deg_rows = _deg_call(dst)
    xw = _mm_call(x, W)
    dis = _dis_call(deg_rows)
    y = _scale_call(xw, dis)
    agg = _agg_call(src, dst, y)
    out = _final_call(agg.reshape(NC, NP, D), y, dis, b)
    return out


# speedup vs baseline: 33.8380x; 1.2911x over previous
import dataclasses

import jax
import jax.numpy as jnp
from jax import lax
from jax.experimental import pallas as pl
from jax.experimental.pallas import tpu as pltpu
from jax.experimental.pallas import tpu_sc as plsc

N = 10000
E = 320000
D = 128

NC = 2
NS = 16
NW = NC * NS
C = 128
NCHUNK = 81
EPW = C * NCHUNK
EPAD = NW * EPW
NPAD_DST = 240
NP = N + NPAD_DST
RPS = NP // NS


def _sc_params():
    cp = pltpu.CompilerParams()
    if "needs_layout_passes" in pltpu.CompilerParams.__dataclass_fields__:
        cp = dataclasses.replace(cp, needs_layout_passes=False)
    return cp


def _mesh():
    return plsc.VectorSubcoreMesh(core_axis_name="c", subcore_axis_name="s",
                                  num_cores=NC, num_subcores=NS)


def _deg_body(sd_hbm, deg_hbm, hist, ib0, ib1, lsem0, lsem1, esem):
    cid = lax.axis_index("c")
    sid = lax.axis_index("s")
    w = cid * NS + sid
    base = w * NCHUNK
    ones = jnp.full((16,), 1.0, jnp.float32)

    @pl.loop(0, NP // 16)
    def _(i):
        hist[pl.ds(i * 16, 16)] = jnp.zeros((16,), jnp.float32)

    def scat(buf):
        @pl.loop(0, 8)
        def _(g):
            plsc.addupdate_scatter(hist, [buf[1, pl.ds(g * 16, 16)]], ones)

    pltpu.sync_copy(sd_hbm.at[base], ib0)

    @pl.loop(0, NCHUNK - 1, step=2)
    def _(j):
        l1 = pltpu.async_copy(sd_hbm.at[base + j + 1], ib1, lsem1)
        scat(ib0)
        l1.wait()
        pltpu.async_copy(sd_hbm.at[base + j + 2], ib0, lsem0)
        scat(ib1)
        pltpu.make_async_copy(sd_hbm.at[base], ib0, lsem0).wait()

    scat(ib0)

    pltpu.async_copy(hist, deg_hbm.at[w], esem).wait()


@jax.jit
def _deg_call(sd):
    k = pl.kernel(
        _deg_body,
        out_type=jax.ShapeDtypeStruct((NW, NP), jnp.float32),
        mesh=_mesh(),
        compiler_params=_sc_params(),
        scratch_types=[
            pltpu.VMEM((NP,), jnp.float32),
            pltpu.VMEM((2, 128), jnp.int32),
            pltpu.VMEM((2, 128), jnp.int32),
            pltpu.SemaphoreType.DMA,
            pltpu.SemaphoreType.DMA,
            pltpu.SemaphoreType.DMA,
        ],
    )
    return k(sd)


def _agg_body(sd_hbm, y_hbm, agg_hbm,
              ib0, ib1, rows0, rows1, agg_sh, gsem0, gsem1):
    cid = lax.axis_index("c")
    sid = lax.axis_index("s")
    base = (cid * NS + sid) * NCHUNK

    @pl.loop(0, C)
    def _(r):
        @pl.loop(0, D // 16)
        def _(q):
            rows0[r, pl.ds(q * 16, 16)] = jnp.zeros((16,), jnp.float32)

    @pl.loop(0, RPS // C)
    def _(t):
        pltpu.sync_copy(rows0, agg_sh.at[pl.ds(sid * RPS + t * C, C)])

    plsc.subcore_barrier()

    def gather(ib, rows, gsem):
        pltpu.async_copy(y_hbm.at[ib.at[0]], rows, gsem)

    def gather_wait(ib, rows, gsem):
        pltpu.make_async_copy(y_hbm.at[ib.at[0]], rows, gsem).wait()

    def scatter(ib, rows):
        pltpu.sync_copy(rows, agg_sh.at[ib.at[1]], add=True)

    pltpu.sync_copy(sd_hbm.at[base], ib0)
    gather(ib0, rows0, gsem0)

    @pl.loop(0, NCHUNK - 1, step=2)
    def _(j):
        pltpu.sync_copy(sd_hbm.at[base + j + 1], ib1)
        gather(ib1, rows1, gsem1)
        gather_wait(ib0, rows0, gsem0)
        scatter(ib0, rows0)
        pltpu.sync_copy(sd_hbm.at[base + j + 2], ib0)
        gather(ib0, rows0, gsem0)
        gather_wait(ib1, rows1, gsem1)
        scatter(ib1, rows1)

    gather_wait(ib0, rows0, gsem0)
    scatter(ib0, rows0)

    plsc.subcore_barrier()
    pltpu.sync_copy(agg_sh.at[pl.ds(sid * RPS, RPS)],
                    agg_hbm.at[pl.ds(cid * NP + sid * RPS, RPS)])


@jax.jit
def _agg_call(sd, y):
    k = pl.kernel(
        _agg_body,
        out_type=jax.ShapeDtypeStruct((NC * NP, D), jnp.float32),
        mesh=_mesh(),
        scratch_types=[
            pltpu.VMEM((2, 128), jnp.int32),
            pltpu.VMEM((2, 128), jnp.int32),
            pltpu.VMEM((C, D), jnp.float32),
            pltpu.VMEM((C, D), jnp.float32),
            pltpu.VMEM_SHARED((NP, D), jnp.float32),
            pltpu.SemaphoreType.DMA,
            pltpu.SemaphoreType.DMA,
        ],
    )
    return k(sd, y)


_RB = 1000


def _mm_body(x_ref, w_ref, o_ref):
    o_ref[...] = jnp.dot(x_ref[...], w_ref[...],
                         preferred_element_type=jnp.float32)


@jax.jit
def _mm_call(x, W):
    return pl.pallas_call(
        _mm_body,
        grid=(N // _RB,),
        in_specs=[
            pl.BlockSpec((_RB, D), lambda i: (i, 0)),
            pl.BlockSpec((D, D), lambda i: (0, 0)),
        ],
        out_specs=pl.BlockSpec((_RB, D), lambda i: (i, 0)),
        out_shape=jax.ShapeDtypeStruct((N, D), jnp.float32),
    )(x, W)


def _dis_body(t_ref, o_ref):
    deg = jnp.sum(t_ref[:, :N], axis=0) + 1.0
    o_ref[...] = lax.rsqrt(deg)[:, None]


@jax.jit
def _dis_call(deg_rows):
    return pl.pallas_call(
        _dis_body,
        grid=(1,),
        in_specs=[pl.BlockSpec((NW, NP), lambda i: (0, 0))],
        out_specs=pl.BlockSpec((N, 1), lambda i: (0, 0)),
        out_shape=jax.ShapeDtypeStruct((N, 1), jnp.float32),
    )(deg_rows)


def _scale_body(xw_ref, dis_ref, o_ref):
    o_ref[...] = xw_ref[...] * dis_ref[...]


@jax.jit
def _scale_call(xw, dis):
    return pl.pallas_call(
        _scale_body,
        grid=(N // _RB,),
        in_specs=[
            pl.BlockSpec((_RB, D), lambda i: (i, 0)),
            pl.BlockSpec((_RB, 1), lambda i: (i, 0)),
        ],
        out_specs=pl.BlockSpec((_RB, D), lambda i: (i, 0)),
        out_shape=jax.ShapeDtypeStruct((N, D), jnp.float32),
    )(xw, dis)


def _final_body(a_ref, y_ref, dis_ref, b_ref, o_ref):
    acc = a_ref[0] + a_ref[1] + y_ref[...]
    o_ref[...] = acc * dis_ref[...] + b_ref[...][None, :]


@jax.jit
def _final_call(agg3, y, dis, b):
    return pl.pallas_call(
        _final_body,
        grid=(N // _RB,),
        in_specs=[
            pl.BlockSpec((NC, _RB, D), lambda i: (0, i, 0)),
            pl.BlockSpec((_RB, D), lambda i: (i, 0)),
            pl.BlockSpec((_RB, 1), lambda i: (i, 0)),
            pl.BlockSpec((D,), lambda i: (0,)),
        ],
        out_specs=pl.BlockSpec((_RB, D), lambda i: (i, 0)),
        out_shape=jax.ShapeDtypeStruct((N, D), jnp.float32),
    )(agg3, y, dis, b)


def kernel(x, edge_index, W, b):
    src = edge_index[0]
    dst = edge_index[1]
    npad = EPAD - E
    pad_src = (jnp.arange(npad, dtype=jnp.int32) * 97) % N
    pad_dst = N + (jnp.arange(npad, dtype=jnp.int32) % NPAD_DST)
    srcp = jnp.concatenate([src, pad_src]).reshape(NW * NCHUNK, 128)
    dstp = jnp.concatenate([dst, pad_dst]).reshape(NW * NCHUNK, 128)
    sd = jnp.stack([srcp, dstp], axis=1)

    deg_rows = _deg_call(sd)
    xw = _mm_call(x, W)
    dis = _dis_call(deg_rows)
    y = _scale_call(xw, dis)
    agg = _agg_call(sd, y)
    out = _final_call(agg.reshape(NC, NP, D), y, dis, b)
    return out


# speedup vs baseline: 41.6612x; 1.2312x over previous
import dataclasses

import jax
import jax.numpy as jnp
from jax import lax
from jax.experimental import pallas as pl
from jax.experimental.pallas import tpu as pltpu
from jax.experimental.pallas import tpu_sc as plsc

N = 10000
E = 320000
D = 128

NC = 2
NS = 16
NW = NC * NS
C = 128
NCHUNK = 81
EPW = C * NCHUNK
EPAD = NW * EPW
NPAD_DST = 240
NP = N + NPAD_DST
RPS = NP // NS


def _sc_params():
    cp = pltpu.CompilerParams()
    if "needs_layout_passes" in pltpu.CompilerParams.__dataclass_fields__:
        cp = dataclasses.replace(cp, needs_layout_passes=False)
    return cp


def _mesh():
    return plsc.VectorSubcoreMesh(core_axis_name="c", subcore_axis_name="s",
                                  num_cores=NC, num_subcores=NS)


EPW_DEG = E // NW


def _deg_body(dst_hbm, deg_hbm, hist, dbuf, lsem, esem):
    cid = lax.axis_index("c")
    sid = lax.axis_index("s")
    w = cid * NS + sid
    ones = jnp.full((16,), 1.0, jnp.float32)

    ld = pltpu.async_copy(dst_hbm.at[pl.ds(w * EPW_DEG, EPW_DEG)], dbuf, lsem)

    @pl.loop(0, N // 16)
    def _(i):
        hist[pl.ds(i * 16, 16)] = jnp.zeros((16,), jnp.float32)

    ld.wait()

    @pl.loop(0, EPW_DEG // 16)
    def _(g):
        plsc.addupdate_scatter(hist, [dbuf[pl.ds(g * 16, 16)]], ones)

    pltpu.async_copy(hist, deg_hbm.at[w], esem).wait()


@jax.jit
def _deg_call(dst):
    k = pl.kernel(
        _deg_body,
        out_type=jax.ShapeDtypeStruct((NW, N), jnp.float32),
        mesh=_mesh(),
        compiler_params=_sc_params(),
        scratch_types=[
            pltpu.VMEM((N,), jnp.float32),
            pltpu.VMEM((EPW_DEG,), jnp.int32),
            pltpu.SemaphoreType.DMA,
            pltpu.SemaphoreType.DMA,
        ],
    )
    return k(dst)


def _agg_body(sd_hbm, y_hbm, agg_hbm,
              ib0, ib1, rows0, rows1, agg_sh, gsem0, gsem1):
    cid = lax.axis_index("c")
    sid = lax.axis_index("s")
    base = (cid * NS + sid) * NCHUNK

    @pl.loop(0, C)
    def _(r):
        @pl.loop(0, D // 16)
        def _(q):
            rows0[r, pl.ds(q * 16, 16)] = jnp.zeros((16,), jnp.float32)

    @pl.loop(0, RPS // C)
    def _(t):
        pltpu.sync_copy(rows0, agg_sh.at[pl.ds(sid * RPS + t * C, C)])

    plsc.subcore_barrier()

    def gather(ib, rows, gsem):
        pltpu.async_copy(y_hbm.at[ib.at[0]], rows, gsem)

    def gather_wait(ib, rows, gsem):
        pltpu.make_async_copy(y_hbm.at[ib.at[0]], rows, gsem).wait()

    def scatter(ib, rows):
        pltpu.sync_copy(rows, agg_sh.at[ib.at[1]], add=True)

    pltpu.sync_copy(sd_hbm.at[base], ib0)
    gather(ib0, rows0, gsem0)

    @pl.loop(0, NCHUNK - 1, step=2)
    def _(j):
        pltpu.sync_copy(sd_hbm.at[base + j + 1], ib1)
        gather(ib1, rows1, gsem1)
        gather_wait(ib0, rows0, gsem0)
        scatter(ib0, rows0)
        pltpu.sync_copy(sd_hbm.at[base + j + 2], ib0)
        gather(ib0, rows0, gsem0)
        gather_wait(ib1, rows1, gsem1)
        scatter(ib1, rows1)

    gather_wait(ib0, rows0, gsem0)
    scatter(ib0, rows0)

    plsc.subcore_barrier()
    pltpu.sync_copy(agg_sh.at[pl.ds(sid * RPS, RPS)],
                    agg_hbm.at[pl.ds(cid * NP + sid * RPS, RPS)])


@jax.jit
def _agg_call(sd, y):
    k = pl.kernel(
        _agg_body,
        out_type=jax.ShapeDtypeStruct((NC * NP, D), jnp.float32),
        mesh=_mesh(),
        scratch_types=[
            pltpu.VMEM((2, 128), jnp.int32),
            pltpu.VMEM((2, 128), jnp.int32),
            pltpu.VMEM((C, D), jnp.float32),
            pltpu.VMEM((C, D), jnp.float32),
            pltpu.VMEM_SHARED((NP, D), jnp.float32),
            pltpu.SemaphoreType.DMA,
            pltpu.SemaphoreType.DMA,
        ],
    )
    return k(sd, y)


_RB = 1000


def _mm_body(x_ref, w_ref, o_ref):
    o_ref[...] = jnp.dot(x_ref[...], w_ref[...],
                         preferred_element_type=jnp.float32)


@jax.jit
def _mm_call(x, W):
    return pl.pallas_call(
        _mm_body,
        grid=(N // _RB,),
        in_specs=[
            pl.BlockSpec((_RB, D), lambda i: (i, 0)),
            pl.BlockSpec((D, D), lambda i: (0, 0)),
        ],
        out_specs=pl.BlockSpec((_RB, D), lambda i: (i, 0)),
        out_shape=jax.ShapeDtypeStruct((N, D), jnp.float32),
    )(x, W)


def _disscale_body(xw_ref, t_ref, y_ref, dis_ref):
    deg = jnp.sum(t_ref[...], axis=0) + 1.0
    dis = lax.rsqrt(deg)
    dis_ref[...] = dis[:, None]
    y_ref[...] = xw_ref[...] * dis[:, None]


@jax.jit
def _disscale_call(xw, deg_rows):
    return pl.pallas_call(
        _disscale_body,
        grid=(1,),
        in_specs=[
            pl.BlockSpec((N, D), lambda i: (0, 0)),
            pl.BlockSpec((NW, N), lambda i: (0, 0)),
        ],
        out_specs=[
            pl.BlockSpec((N, D), lambda i: (0, 0)),
            pl.BlockSpec((N, 1), lambda i: (0, 0)),
        ],
        out_shape=[
            jax.ShapeDtypeStruct((N, D), jnp.float32),
            jax.ShapeDtypeStruct((N, 1), jnp.float32),
        ],
    )(xw, deg_rows)


def _final_body(a_ref, y_ref, dis_ref, b_ref, o_ref):
    acc = a_ref[0] + a_ref[1] + y_ref[...]
    o_ref[...] = acc * dis_ref[...] + b_ref[...][None, :]


@jax.jit
def _final_call(agg3, y, dis, b):
    return pl.pallas_call(
        _final_body,
        grid=(N // _RB,),
        in_specs=[
            pl.BlockSpec((NC, _RB, D), lambda i: (0, i, 0)),
            pl.BlockSpec((_RB, D), lambda i: (i, 0)),
            pl.BlockSpec((_RB, 1), lambda i: (i, 0)),
            pl.BlockSpec((D,), lambda i: (0,)),
        ],
        out_specs=pl.BlockSpec((_RB, D), lambda i: (i, 0)),
        out_shape=jax.ShapeDtypeStruct((N, D), jnp.float32),
    )(agg3, y, dis, b)


def kernel(x, edge_index, W, b):
    src = edge_index[0]
    dst = edge_index[1]
    npad = EPAD - E
    pad_src = (jnp.arange(npad, dtype=jnp.int32) * 97) % N
    pad_dst = N + (jnp.arange(npad, dtype=jnp.int32) % NPAD_DST)
    srcp = jnp.concatenate([src, pad_src]).reshape(NW * NCHUNK, 128)
    dstp = jnp.concatenate([dst, pad_dst]).reshape(NW * NCHUNK, 128)
    sd = jnp.stack([srcp, dstp], axis=1)

    deg_rows = _deg_call(dst)
    xw = _mm_call(x, W)
    y, dis = _disscale_call(xw, deg_rows)
    agg = _agg_call(sd, y)
    out = _final_call(agg.reshape(NC, NP, D), y, dis, b)
    return out


# speedup vs baseline: 44.9113x; 1.0780x over previous
import dataclasses

import jax
import jax.numpy as jnp
from jax import lax
from jax.experimental import pallas as pl
from jax.experimental.pallas import tpu as pltpu
from jax.experimental.pallas import tpu_sc as plsc

N = 10000
E = 320000
D = 128

NC = 2
NS = 16
NW = NC * NS
C = 128
NCHUNK = 81
EPW = C * NCHUNK
EPAD = NW * EPW
NPAD_DST = 240
NP = N + NPAD_DST
RPS = NP // NS


def _sc_params():
    cp = pltpu.CompilerParams()
    if "needs_layout_passes" in pltpu.CompilerParams.__dataclass_fields__:
        cp = dataclasses.replace(cp, needs_layout_passes=False)
    return cp


def _mesh():
    return plsc.VectorSubcoreMesh(core_axis_name="c", subcore_axis_name="s",
                                  num_cores=NC, num_subcores=NS)


EPW_DEG = E // NW


def _deg_body(dst_hbm, deg_hbm, hist, dbuf, lsem, esem):
    cid = lax.axis_index("c")
    sid = lax.axis_index("s")
    w = cid * NS + sid
    ones = jnp.full((16,), 1.0, jnp.float32)

    ld = pltpu.async_copy(dst_hbm.at[pl.ds(w * EPW_DEG, EPW_DEG)], dbuf, lsem)

    @pl.loop(0, N // 16)
    def _(i):
        hist[pl.ds(i * 16, 16)] = jnp.zeros((16,), jnp.float32)

    ld.wait()

    @pl.loop(0, EPW_DEG // 16)
    def _(g):
        plsc.addupdate_scatter(hist, [dbuf[pl.ds(g * 16, 16)]], ones)

    pltpu.async_copy(hist, deg_hbm.at[w], esem).wait()


@jax.jit
def _deg_call(dst):
    k = pl.kernel(
        _deg_body,
        out_type=jax.ShapeDtypeStruct((NW, N), jnp.float32),
        mesh=_mesh(),
        compiler_params=_sc_params(),
        scratch_types=[
            pltpu.VMEM((N,), jnp.float32),
            pltpu.VMEM((EPW_DEG,), jnp.int32),
            pltpu.SemaphoreType.DMA,
            pltpu.SemaphoreType.DMA,
        ],
    )
    return k(dst)


def _agg_body(sd_hbm, y_hbm, agg_hbm,
              ib0, ib1, ib2, ib3, rows0, rows1, agg_sh,
              lsem0, lsem1, lsem2, lsem3, gsem0, gsem1):
    cid = lax.axis_index("c")
    sid = lax.axis_index("s")
    base = (cid * NS + sid) * NCHUNK

    ibs = (ib0, ib1, ib2, ib3)
    lsems = (lsem0, lsem1, lsem2, lsem3)
    rows = (rows0, rows1)
    gsems = (gsem0, gsem1)

    l0 = pltpu.async_copy(sd_hbm.at[base], ib0, lsem0)
    pltpu.async_copy(sd_hbm.at[base + 1], ib1, lsem1)
    pltpu.async_copy(sd_hbm.at[base + 2], ib2, lsem2)

    @pl.loop(0, C)
    def _(r):
        @pl.loop(0, D // 16)
        def _(q):
            rows0[r, pl.ds(q * 16, 16)] = jnp.zeros((16,), jnp.float32)

    @pl.loop(0, RPS // C)
    def _(t):
        pltpu.sync_copy(rows0, agg_sh.at[pl.ds(sid * RPS + t * C, C)])

    l0.wait()
    pltpu.async_copy(y_hbm.at[ib0.at[0]], rows0, gsem0)
    pltpu.async_copy(sd_hbm.at[base + 3], ib3, lsem3)
    plsc.subcore_barrier()

    @pl.loop(1, NCHUNK, step=4)
    def _(j):
        for k in range(4):
            m = j + k
            s = (1 + k) % 4
            sp = k % 4
            pltpu.make_async_copy(sd_hbm.at[base + m], ibs[s], lsems[s]).wait()
            pltpu.async_copy(y_hbm.at[ibs[s].at[0]], rows[(1 + k) % 2],
                             gsems[(1 + k) % 2])
            pltpu.make_async_copy(y_hbm.at[ibs[sp].at[0]], rows[k % 2],
                                  gsems[k % 2]).wait()
            pltpu.sync_copy(rows[k % 2], agg_sh.at[ibs[sp].at[1]], add=True)
            pltpu.async_copy(sd_hbm.at[base + m + 3], ibs[sp], lsems[sp])

    pltpu.make_async_copy(y_hbm.at[ib0.at[0]], rows0, gsem0).wait()
    pltpu.sync_copy(rows0, agg_sh.at[ib0.at[1]], add=True)
    for k in range(3):
        pltpu.make_async_copy(sd_hbm.at[base + NCHUNK + k],
                              (ib1, ib2, ib3)[k],
                              (lsem1, lsem2, lsem3)[k]).wait()

    plsc.subcore_barrier()
    pltpu.sync_copy(agg_sh.at[pl.ds(sid * RPS, RPS)],
                    agg_hbm.at[pl.ds(cid * NP + sid * RPS, RPS)])


@jax.jit
def _agg_call(sd, y):
    k = pl.kernel(
        _agg_body,
        out_type=jax.ShapeDtypeStruct((NC * NP, D), jnp.float32),
        mesh=_mesh(),
        scratch_types=[
            pltpu.VMEM((2, 128), jnp.int32),
            pltpu.VMEM((2, 128), jnp.int32),
            pltpu.VMEM((2, 128), jnp.int32),
            pltpu.VMEM((2, 128), jnp.int32),
            pltpu.VMEM((C, D), jnp.float32),
            pltpu.VMEM((C, D), jnp.float32),
            pltpu.VMEM_SHARED((NP, D), jnp.float32),
            pltpu.SemaphoreType.DMA,
            pltpu.SemaphoreType.DMA,
            pltpu.SemaphoreType.DMA,
            pltpu.SemaphoreType.DMA,
            pltpu.SemaphoreType.DMA,
            pltpu.SemaphoreType.DMA,
        ],
    )
    return k(sd, y)


_RB = 1000


def _mm_body(x_ref, w_ref, o_ref):
    o_ref[...] = jnp.dot(x_ref[...], w_ref[...],
                         preferred_element_type=jnp.float32)


@jax.jit
def _mm_call(x, W):
    return pl.pallas_call(
        _mm_body,
        grid=(N // _RB,),
        in_specs=[
            pl.BlockSpec((_RB, D), lambda i: (i, 0)),
            pl.BlockSpec((D, D), lambda i: (0, 0)),
        ],
        out_specs=pl.BlockSpec((_RB, D), lambda i: (i, 0)),
        out_shape=jax.ShapeDtypeStruct((N, D), jnp.float32),
    )(x, W)


def _disscale_body(xw_ref, t_ref, y_ref, dis_ref):
    deg = jnp.sum(t_ref[...], axis=0) + 1.0
    dis = lax.rsqrt(deg)
    dis_ref[...] = dis[:, None]
    y_ref[...] = xw_ref[...] * dis[:, None]


@jax.jit
def _disscale_call(xw, deg_rows):
    return pl.pallas_call(
        _disscale_body,
        grid=(1,),
        in_specs=[
            pl.BlockSpec((N, D), lambda i: (0, 0)),
            pl.BlockSpec((NW, N), lambda i: (0, 0)),
        ],
        out_specs=[
            pl.BlockSpec((N, D), lambda i: (0, 0)),
            pl.BlockSpec((N, 1), lambda i: (0, 0)),
        ],
        out_shape=[
            jax.ShapeDtypeStruct((N, D), jnp.float32),
            jax.ShapeDtypeStruct((N, 1), jnp.float32),
        ],
    )(xw, deg_rows)


def _final_body(a_ref, y_ref, dis_ref, b_ref, o_ref):
    acc = a_ref[0] + a_ref[1] + y_ref[...]
    o_ref[...] = acc * dis_ref[...] + b_ref[...][None, :]


@jax.jit
def _final_call(agg3, y, dis, b):
    return pl.pallas_call(
        _final_body,
        grid=(N // _RB,),
        in_specs=[
            pl.BlockSpec((NC, _RB, D), lambda i: (0, i, 0)),
            pl.BlockSpec((_RB, D), lambda i: (i, 0)),
            pl.BlockSpec((_RB, 1), lambda i: (i, 0)),
            pl.BlockSpec((D,), lambda i: (0,)),
        ],
        out_specs=pl.BlockSpec((_RB, D), lambda i: (i, 0)),
        out_shape=jax.ShapeDtypeStruct((N, D), jnp.float32),
    )(agg3, y, dis, b)


def kernel(x, edge_index, W, b):
    src = edge_index[0]
    dst = edge_index[1]
    npad = EPAD - E
    pad_src = (jnp.arange(npad, dtype=jnp.int32) * 97) % N
    pad_dst = N + (jnp.arange(npad, dtype=jnp.int32) % NPAD_DST)
    srcp = jnp.concatenate([src, pad_src]).reshape(NW * NCHUNK, 128)
    dstp = jnp.concatenate([dst, pad_dst]).reshape(NW * NCHUNK, 128)
    sd = jnp.stack([srcp, dstp], axis=1)
    sd = jnp.concatenate([sd, jnp.zeros((3, 2, 128), jnp.int32)])

    deg_rows = _deg_call(dst)
    xw = _mm_call(x, W)
    y, dis = _disscale_call(xw, deg_rows)
    agg = _agg_call(sd, y)
    out = _final_call(agg.reshape(NC, NP, D), y, dis, b)
    return out


# speedup vs baseline: 46.9443x; 1.0453x over previous
import dataclasses

import jax
import jax.numpy as jnp
from jax import lax
from jax.experimental import pallas as pl
from jax.experimental.pallas import tpu as pltpu
from jax.experimental.pallas import tpu_sc as plsc

N = 10000
E = 320000
D = 128

NC = 2
NS = 16
NW = NC * NS
C = 128
NCHUNK = 81
EPW = C * NCHUNK
EPAD = NW * EPW
NPAD_DST = 240
NP = N + NPAD_DST
RPS = NP // NS


def _sc_params():
    cp = pltpu.CompilerParams()
    if "needs_layout_passes" in pltpu.CompilerParams.__dataclass_fields__:
        cp = dataclasses.replace(cp, needs_layout_passes=False)
    return cp


def _mesh():
    return plsc.VectorSubcoreMesh(core_axis_name="c", subcore_axis_name="s",
                                  num_cores=NC, num_subcores=NS)


EPW_DEG = E // NW


def _deg_body(dst_hbm, deg_hbm, hist, dbuf, lsem, esem):
    cid = lax.axis_index("c")
    sid = lax.axis_index("s")
    w = cid * NS + sid
    ones = jnp.full((16,), 1.0, jnp.float32)

    ld = pltpu.async_copy(dst_hbm.at[pl.ds(w * EPW_DEG, EPW_DEG)], dbuf, lsem)

    @pl.loop(0, N // 16)
    def _(i):
        hist[pl.ds(i * 16, 16)] = jnp.zeros((16,), jnp.float32)

    ld.wait()

    @pl.loop(0, EPW_DEG // 16)
    def _(g):
        plsc.addupdate_scatter(hist, [dbuf[pl.ds(g * 16, 16)]], ones)

    pltpu.async_copy(hist, deg_hbm.at[w], esem).wait()


@jax.jit
def _deg_call(dst):
    k = pl.kernel(
        _deg_body,
        out_type=jax.ShapeDtypeStruct((NW, N), jnp.float32),
        mesh=_mesh(),
        compiler_params=_sc_params(),
        scratch_types=[
            pltpu.VMEM((N,), jnp.float32),
            pltpu.VMEM((EPW_DEG,), jnp.int32),
            pltpu.SemaphoreType.DMA,
            pltpu.SemaphoreType.DMA,
        ],
    )
    return k(dst)


def _agg_body(sd_hbm, y_hbm, agg_hbm,
              ib0, ib1, ib2, ib3, rows0, rows1, agg_sh,
              lsem0, lsem1, lsem2, lsem3, gsem0, gsem1):
    cid = lax.axis_index("c")
    sid = lax.axis_index("s")
    base = (cid * NS + sid) * NCHUNK

    ibs = (ib0, ib1, ib2, ib3)
    lsems = (lsem0, lsem1, lsem2, lsem3)
    rows = (rows0, rows1)
    gsems = (gsem0, gsem1)

    l0 = pltpu.async_copy(sd_hbm.at[base], ib0, lsem0)
    pltpu.async_copy(sd_hbm.at[base + 1], ib1, lsem1)
    pltpu.async_copy(sd_hbm.at[base + 2], ib2, lsem2)

    @pl.loop(0, C)
    def _(r):
        @pl.loop(0, D // 16)
        def _(q):
            rows0[r, pl.ds(q * 16, 16)] = jnp.zeros((16,), jnp.float32)

    @pl.loop(0, RPS // C)
    def _(t):
        pltpu.sync_copy(rows0, agg_sh.at[pl.ds(sid * RPS + t * C, C)])

    l0.wait()
    pltpu.async_copy(y_hbm.at[ib0.at[0]], rows0, gsem0)
    pltpu.async_copy(sd_hbm.at[base + 3], ib3, lsem3)
    plsc.subcore_barrier()

    @pl.loop(1, NCHUNK, step=4)
    def _(j):
        for k in range(4):
            m = j + k
            s = (1 + k) % 4
            sp = k % 4
            pltpu.make_async_copy(sd_hbm.at[base + m], ibs[s], lsems[s]).wait()
            pltpu.async_copy(y_hbm.at[ibs[s].at[0]], rows[(1 + k) % 2],
                             gsems[(1 + k) % 2])
            pltpu.make_async_copy(y_hbm.at[ibs[sp].at[0]], rows[k % 2],
                                  gsems[k % 2]).wait()
            pltpu.sync_copy(rows[k % 2], agg_sh.at[ibs[sp].at[1]], add=True)
            pltpu.async_copy(sd_hbm.at[base + m + 3], ibs[sp], lsems[sp])

    pltpu.make_async_copy(y_hbm.at[ib0.at[0]], rows0, gsem0).wait()
    pltpu.sync_copy(rows0, agg_sh.at[ib0.at[1]], add=True)
    for k in range(3):
        pltpu.make_async_copy(sd_hbm.at[base + NCHUNK + k],
                              (ib1, ib2, ib3)[k],
                              (lsem1, lsem2, lsem3)[k]).wait()

    plsc.subcore_barrier()
    pltpu.sync_copy(agg_sh.at[pl.ds(sid * RPS, RPS)],
                    agg_hbm.at[pl.ds(cid * NP + sid * RPS, RPS)])


@jax.jit
def _agg_call(sd, y):
    k = pl.kernel(
        _agg_body,
        out_type=jax.ShapeDtypeStruct((NC * NP, D), jnp.float32),
        mesh=_mesh(),
        scratch_types=[
            pltpu.VMEM((2, 128), jnp.int32),
            pltpu.VMEM((2, 128), jnp.int32),
            pltpu.VMEM((2, 128), jnp.int32),
            pltpu.VMEM((2, 128), jnp.int32),
            pltpu.VMEM((C, D), jnp.float32),
            pltpu.VMEM((C, D), jnp.float32),
            pltpu.VMEM_SHARED((NP, D), jnp.float32),
            pltpu.SemaphoreType.DMA,
            pltpu.SemaphoreType.DMA,
            pltpu.SemaphoreType.DMA,
            pltpu.SemaphoreType.DMA,
            pltpu.SemaphoreType.DMA,
            pltpu.SemaphoreType.DMA,
        ],
    )
    return k(sd, y)


_RB = 1000


def _disscale_body(x_ref, w_ref, t_ref, y_ref, dis_ref):
    deg = jnp.sum(t_ref[...], axis=0) + 1.0
    dis = lax.rsqrt(deg)
    dis_ref[...] = dis[:, None]
    xw = jnp.dot(x_ref[...], w_ref[...], preferred_element_type=jnp.float32)
    y_ref[...] = xw * dis[:, None]


@jax.jit
def _disscale_call(x, W, deg_rows):
    return pl.pallas_call(
        _disscale_body,
        grid=(1,),
        in_specs=[
            pl.BlockSpec((N, D), lambda i: (0, 0)),
            pl.BlockSpec((D, D), lambda i: (0, 0)),
            pl.BlockSpec((NW, N), lambda i: (0, 0)),
        ],
        out_specs=[
            pl.BlockSpec((N, D), lambda i: (0, 0)),
            pl.BlockSpec((N, 1), lambda i: (0, 0)),
        ],
        out_shape=[
            jax.ShapeDtypeStruct((N, D), jnp.float32),
            jax.ShapeDtypeStruct((N, 1), jnp.float32),
        ],
    )(x, W, deg_rows)


def _final_body(a_ref, y_ref, dis_ref, b_ref, o_ref):
    acc = a_ref[0] + a_ref[1] + y_ref[...]
    o_ref[...] = acc * dis_ref[...] + b_ref[...][None, :]


@jax.jit
def _final_call(agg3, y, dis, b):
    return pl.pallas_call(
        _final_body,
        grid=(N // _RB,),
        in_specs=[
            pl.BlockSpec((NC, _RB, D), lambda i: (0, i, 0)),
            pl.BlockSpec((_RB, D), lambda i: (i, 0)),
            pl.BlockSpec((_RB, 1), lambda i: (i, 0)),
            pl.BlockSpec((D,), lambda i: (0,)),
        ],
        out_specs=pl.BlockSpec((_RB, D), lambda i: (i, 0)),
        out_shape=jax.ShapeDtypeStruct((N, D), jnp.float32),
    )(agg3, y, dis, b)


def kernel(x, edge_index, W, b):
    src = edge_index[0]
    dst = edge_index[1]
    npad = EPAD - E
    pad_src = (jnp.arange(npad, dtype=jnp.int32) * 97) % N
    pad_dst = N + (jnp.arange(npad, dtype=jnp.int32) % NPAD_DST)
    srcp = jnp.concatenate([src, pad_src]).reshape(NW * NCHUNK, 128)
    dstp = jnp.concatenate([dst, pad_dst]).reshape(NW * NCHUNK, 128)
    sd = jnp.stack([srcp, dstp], axis=1)
    sd = jnp.concatenate([sd, jnp.zeros((3, 2, 128), jnp.int32)])

    deg_rows = _deg_call(dst)
    y, dis = _disscale_call(x, W, deg_rows)
    agg = _agg_call(sd, y)
    out = _final_call(agg.reshape(NC, NP, D), y, dis, b)
    return out
